# baseline (speedup 1.0000x reference)
import functools

import jax
import jax.numpy as jnp
from jax import lax
from jax.experimental import pallas as pl
from jax.experimental.pallas import tpu as pltpu
from jax.experimental.pallas import tpu_sc as plsc

L = 200
D = 32
NC, NS = 2, 16
NW = NC * NS

CH = 1024
N_G = CH // 128

TBK = 2048


def _transpose_table(token_table):
    vocab = token_table.shape[0]

    def tbody(in_ref, out_ref):
        blk = in_ref[...]
        t = jnp.transpose(blk, (1, 0))
        t3 = t.reshape(TBK // 4, 4, D)
        for a in range(4):
            out_ref[:, a * D:(a + 1) * D] = t3[:, a, :]

    tok128 = pl.pallas_call(
        tbody,
        out_shape=jax.ShapeDtypeStruct((vocab // 4, 4 * D), jnp.float32),
        grid=((vocab + TBK - 1) // TBK,),
        in_specs=[pl.BlockSpec((D, TBK), lambda j: (0, j))],
        out_specs=pl.BlockSpec((TBK // 4, 4 * D), lambda j: (j, 0)),
    )(token_table.T)
    return tok128.reshape(vocab, D)


def _emb_call(x2d, tok2, pos_table, total_rows):
    rows_per_w = total_rows // NW
    n_chunks = rows_per_w // CH
    idx_rows_w = rows_per_w // 128
    mesh = plsc.VectorSubcoreMesh(core_axis_name="c", subcore_axis_name="s")

    @functools.partial(
        pl.kernel,
        out_type=jax.ShapeDtypeStruct((total_rows, D), jnp.float32),
        mesh=mesh,
        compiler_params=pltpu.CompilerParams(use_tc_tiling_on_sc=False),
        scratch_types=[
            pltpu.VMEM((idx_rows_w, 128), jnp.int32),
            pltpu.VMEM((CH, D), jnp.float32),
            pltpu.VMEM((CH, D), jnp.float32),
            pltpu.VMEM((L, D), jnp.float32),
            pltpu.SemaphoreType.DMA,
            pltpu.SemaphoreType.DMA,
            pltpu.SemaphoreType.DMA,
            pltpu.SemaphoreType.DMA,
        ],
    )
    def body(x_hbm, tok_hbm, pos_hbm, out_hbm, idx_v, r0, r1, pos_v,
             sg0, sg1, st0, st1):
        rows = (r0, r1)
        sem_g = (sg0, sg1)
        sem_st = (st0, st1)
        wid = lax.axis_index("s") * NC + lax.axis_index("c")
        base = wid * rows_per_w
        pltpu.sync_copy(x_hbm.at[pl.ds(wid * idx_rows_w, idx_rows_w)], idx_v)
        pltpu.sync_copy(pos_hbm, pos_v)

        def fire_gathers(c, t):
            for j in range(N_G):
                pltpu.async_copy(
                    tok_hbm.at[idx_v.at[c * N_G + j]],
                    rows[t].at[pl.ds(j * 128, 128)],
                    sem_g[t],
                )

        def drain_gathers(c, s):
            for j in range(N_G):
                pltpu.make_async_copy(
                    tok_hbm.at[idx_v.at[c * N_G + j]],
                    rows[s].at[pl.ds(j * 128, 128)],
                    sem_g[s],
                ).wait()

        def add_pos(c, s):
            phase0 = lax.rem(base + c * CH, L)

            def row(rr, p):
                for h in (0, 16):
                    rows[s][rr, pl.ds(h, 16)] = (
                        rows[s][rr, pl.ds(h, 16)] + pos_v[p, pl.ds(h, 16)]
                    )
                p = p + 1
                return lax.select(p == L, 0, p)

            lax.fori_loop(0, CH, row, phase0)

        def store(c, s):
            pltpu.async_copy(
                rows[s],
                out_hbm.at[pl.ds(base + c * CH, CH)],
                sem_st[s],
            )

        def wait_store(c, s):
            pltpu.make_async_copy(
                rows[s],
                out_hbm.at[pl.ds(base + c * CH, CH)],
                sem_st[s],
            ).wait()

        def _step(c, s, t):
            @pl.when(c + 1 < n_chunks)
            def _prefetch():
                @pl.when(c >= 1)
                def _reuse_guard():
                    wait_store(c - 1, t)

                fire_gathers(c + 1, t)

            drain_gathers(c, s)
            add_pos(c, s)
            store(c, s)

        fire_gathers(0, 0)

        @pl.loop(0, n_chunks)
        def chunks(c):
            @pl.when(lax.rem(c, 2) == 0)
            def _even():
                _step(c, 0, 1)

            @pl.when(lax.rem(c, 2) == 1)
            def _odd():
                _step(c, 1, 0)

        wait_store(n_chunks - 2, (n_chunks - 2) % 2)
        wait_store(n_chunks - 1, (n_chunks - 1) % 2)

    return body(x2d, tok2, pos_table)


def kernel(x, token_table, pos_table):
    batch, maxlen = x.shape
    total_rows = batch * maxlen
    x2d = x.reshape(total_rows // 128, 128)
    tok2 = _transpose_table(token_table)
    out = _emb_call(x2d, tok2, pos_table, total_rows)
    return out.reshape(batch, maxlen, D)
